# baseline (speedup 1.0000x reference)
import functools

import jax
import jax.numpy as jnp
from jax import lax
from jax.experimental import pallas as pl
from jax.experimental.pallas import tpu as pltpu
from jax.experimental.pallas import tpu_sc as plsc

N = 10000
E = 320000
H = 128
C = 3
L_MID = 12

NC = 2
NS = 16
NW = NC * NS
EPW = E // NW
K = 80
NCH = EPW // K
RPW = 624
TAIL = N - NS * RPW
NG = H // 16

_mesh = plsc.VectorSubcoreMesh(core_axis_name="c", subcore_axis_name="s")


def _copy_rows(sid, get_src, get_dst):
    pltpu.sync_copy(get_src(sid * RPW, RPW), get_dst(sid * RPW, RPW))

    @pl.when(sid == NS - 1)
    def _():
        pltpu.sync_copy(get_src(NS * RPW, TAIL), get_dst(NS * RPW, TAIL))


@functools.partial(
    pl.kernel,
    out_type=jax.ShapeDtypeStruct((NC, N, H), jnp.float32),
    mesh=_mesh,
    scratch_types=[
        pltpu.VMEM_SHARED((N, H), jnp.float32),
        pltpu.VMEM((3, K), jnp.int32),
        pltpu.VMEM((3, K), jnp.int32),
        pltpu.VMEM((K, H), jnp.float32),
        pltpu.VMEM((K, H), jnp.float32),
        pltpu.VMEM((K, 16), jnp.float32),
        pltpu.VMEM((K, 16), jnp.float32),
        pltpu.SemaphoreType.DMA,
        pltpu.SemaphoreType.DMA,
        pltpu.SemaphoreType.DMA,
        pltpu.SemaphoreType.DMA,
    ],
)
def _agg_sc(q_hbm, sd5_hbm, ew16_hbm, zeros_hbm, out_hbm,
            acc_sh, idx_a, idx_b, rows_a, rows_b, ewb_a, ewb_b,
            isem_a, isem_b, gsem_a, gsem_b):
    cid = lax.axis_index("c")
    sid = lax.axis_index("s")
    wid = cid * NS + sid
    _copy_rows(sid, lambda o, n: zeros_hbm.at[pl.ds(o, n)],
               lambda o, n: acc_sh.at[pl.ds(o, n)])
    plsc.subcore_barrier()

    pltpu.async_copy(sd5_hbm.at[wid, 0], idx_a, isem_a)
    pltpu.async_copy(ew16_hbm.at[wid, 0], ewb_a, isem_a)
    pltpu.make_async_copy(sd5_hbm.at[wid, 0], idx_a, isem_a).wait()
    pltpu.make_async_copy(ew16_hbm.at[wid, 0], ewb_a, isem_a).wait()
    pltpu.async_copy(sd5_hbm.at[wid, 1], idx_b, isem_b)
    pltpu.async_copy(ew16_hbm.at[wid, 1], ewb_b, isem_b)
    pltpu.async_copy(q_hbm.at[idx_a.at[0]], rows_a, gsem_a)

    def scale_rows(ewb_cur, rows_cur):
        def row_body(r, carry):
            w = ewb_cur[r]
            for g in range(NG):
                v = rows_cur[r, pl.ds(g * 16, 16)]
                rows_cur[r, pl.ds(g * 16, 16)] = v * w
            return carry

        lax.fori_loop(0, K, row_body, 0)

    def step(ch, idx_cur, ewb_cur, isem_cur, gsem_cur, rows_cur,
             idx_nxt, ewb_nxt, isem_nxt, gsem_nxt, rows_nxt):
        pltpu.make_async_copy(q_hbm.at[idx_cur.at[0]], rows_cur,
                              gsem_cur).wait()

        @pl.when(ch + 1 < NCH)
        def _():
            pltpu.make_async_copy(sd5_hbm.at[wid, ch + 1], idx_nxt,
                                  isem_nxt).wait()
            pltpu.make_async_copy(ew16_hbm.at[wid, ch + 1], ewb_nxt,
                                  isem_nxt).wait()
            pltpu.async_copy(q_hbm.at[idx_nxt.at[0]], rows_nxt, gsem_nxt)

        scale_rows(ewb_cur, rows_cur)
        pltpu.sync_copy(rows_cur, acc_sh.at[idx_cur.at[1]], add=True)

        @pl.when(ch + 2 < NCH)
        def _():
            pltpu.async_copy(sd5_hbm.at[wid, ch + 2], idx_cur, isem_cur)
            pltpu.async_copy(ew16_hbm.at[wid, ch + 2], ewb_cur, isem_cur)

    def body(ch, carry):
        @pl.when(lax.rem(ch, 2) == 0)
        def _():
            step(ch, idx_a, ewb_a, isem_a, gsem_a, rows_a,
                 idx_b, ewb_b, isem_b, gsem_b, rows_b)

        @pl.when(lax.rem(ch, 2) == 1)
        def _():
            step(ch, idx_b, ewb_b, isem_b, gsem_b, rows_b,
                 idx_a, ewb_a, isem_a, gsem_a, rows_a)

        return carry

    lax.fori_loop(0, NCH, body, 0)
    plsc.subcore_barrier()
    _copy_rows(sid, lambda o, n: acc_sh.at[pl.ds(o, n)],
               lambda o, n: out_hbm.at[cid, pl.ds(o, n)])


@functools.partial(
    pl.kernel,
    out_type=jax.ShapeDtypeStruct((NW, NCH, K, 16), jnp.float32),
    mesh=_mesh,
    scratch_types=[
        pltpu.VMEM((3, K), jnp.int32),
        pltpu.VMEM((K, H), jnp.float32),
        pltpu.VMEM((K, H), jnp.float32),
        pltpu.VMEM((K, 16), jnp.float32),
        pltpu.SemaphoreType.DMA,
    ],
)
def _pew_sc(degsum_hbm, sd5_hbm, out_hbm, idx_v, rs_v, rd_v, p_v, sem):
    cid = lax.axis_index("c")
    sid = lax.axis_index("s")
    wid = cid * NS + sid
    one = jnp.full((16,), 1.0, jnp.float32)

    def chunk(ch, carry):
        pltpu.sync_copy(sd5_hbm.at[wid, ch], idx_v)
        pltpu.async_copy(degsum_hbm.at[idx_v.at[0]], rs_v, sem).wait()
        pltpu.async_copy(degsum_hbm.at[idx_v.at[1]], rd_v, sem).wait()

        def row_body(r, c):
            ds_ = jnp.maximum(rs_v[r, pl.ds(0, 16)], one)
            dd_ = jnp.maximum(rd_v[r, pl.ds(0, 16)], one)
            p_v[r] = ds_ * dd_
            return c

        lax.fori_loop(0, K, row_body, 0)
        pltpu.sync_copy(p_v, out_hbm.at[wid, ch])
        return carry

    lax.fori_loop(0, NCH, chunk, 0)


BN = 1000
GRID = N // BN


def _degsum_body(deg_ref, out_ref):
    out_ref[...] = deg_ref[0] + deg_ref[1]


_degsum_tc = pl.pallas_call(
    _degsum_body,
    grid=(GRID,),
    in_specs=[pl.BlockSpec((NC, BN, H), lambda i: (0, i, 0))],
    out_specs=pl.BlockSpec((BN, H), lambda i: (i, 0)),
    out_shape=jax.ShapeDtypeStruct((N, H), jnp.float32),
)


def _layer_body(h_ref, s_ref, ws_ref, wn_ref, b_ref, h_out_ref, *, act):
    sb = s_ref[0] + s_ref[1]
    z = (jnp.dot(h_ref[...], ws_ref[...], preferred_element_type=jnp.float32)
         + jnp.dot(sb, wn_ref[...], preferred_element_type=jnp.float32)
         + b_ref[...])
    if act:
        z = jnp.maximum(z, 0.0)
    h_out_ref[...] = z


def _make_layer_tc(act):
    return pl.pallas_call(
        functools.partial(_layer_body, act=act),
        grid=(GRID,),
        in_specs=[
            pl.BlockSpec((BN, H), lambda i: (i, 0)),
            pl.BlockSpec((NC, BN, H), lambda i: (0, i, 0)),
            pl.BlockSpec((H, H), lambda i: (0, 0)),
            pl.BlockSpec((H, H), lambda i: (0, 0)),
            pl.BlockSpec((1, H), lambda i: (0, 0)),
        ],
        out_specs=pl.BlockSpec((BN, H), lambda i: (i, 0)),
        out_shape=jax.ShapeDtypeStruct((N, H), jnp.float32),
    )


_layer_tc = _make_layer_tc(act=True)
_final_tc = _make_layer_tc(act=False)


def kernel(x, edge_index, W_in_self, W_in_neigh, b_in,
           W_mid_self, W_mid_neigh, b_mid,
           W_out_self, W_out_neigh, b_out):
    sd3 = (edge_index.astype(jnp.int32)
           .reshape(2, NW, NCH, K).transpose(1, 2, 0, 3))
    one_bits = jnp.full((NW, NCH, 1, K),
                        jnp.float32(1.0).view(jnp.int32), jnp.int32)
    sd5_ones = jnp.concatenate([sd3, one_bits], axis=2)
    onesNH = jnp.ones((N, H), jnp.float32)
    zerosH = jnp.zeros((N, H), jnp.float32)

    ones16e = jnp.ones((NW, NCH, K, 16), jnp.float32)
    deg2 = _agg_sc(onesNH, sd5_ones, ones16e, zerosH)
    degsum = _degsum_tc(deg2)
    p16 = _pew_sc(degsum, sd5_ones)
    ew16 = 1.0 / jnp.sqrt(p16)

    Wso = jnp.zeros((H, H), jnp.float32).at[:, :C].set(W_out_self)
    Wno = jnp.zeros((H, H), jnp.float32).at[:, :C].set(W_out_neigh)
    bo = jnp.zeros((1, H), jnp.float32).at[0, :C].set(b_out)

    h = x
    for li in range(L_MID + 2):
        s = _agg_sc(h, sd5_ones, ew16, zerosH)
        if li == 0:
            h = _layer_tc(h, s, W_in_self, W_in_neigh, b_in.reshape(1, H))
        elif li <= L_MID:
            h = _layer_tc(h, s, W_mid_self[li - 1], W_mid_neigh[li - 1],
                          b_mid[li - 1].reshape(1, H))
        else:
            h = _final_tc(h, s, Wso, Wno, bo)
    return h[:, :C]


# speedup vs baseline: 1.0012x; 1.0012x over previous
import functools

import jax
import jax.numpy as jnp
from jax import lax
from jax.experimental import pallas as pl
from jax.experimental.pallas import tpu as pltpu
from jax.experimental.pallas import tpu_sc as plsc

N = 10000
E = 320000
H = 128
C = 3
L_MID = 12

NC = 2
NS = 16
NW = NC * NS
EPW = E // NW
K = 80
NCH = EPW // K
RPW = 624
TAIL = N - NS * RPW
NG = H // 16

_mesh = plsc.VectorSubcoreMesh(core_axis_name="c", subcore_axis_name="s")


def _copy_rows(sid, get_src, get_dst):
    pltpu.sync_copy(get_src(sid * RPW, RPW), get_dst(sid * RPW, RPW))

    @pl.when(sid == NS - 1)
    def _():
        pltpu.sync_copy(get_src(NS * RPW, TAIL), get_dst(NS * RPW, TAIL))


@functools.partial(
    pl.kernel,
    out_type=jax.ShapeDtypeStruct((NC, N, H), jnp.float32),
    mesh=_mesh,
    scratch_types=[
        pltpu.VMEM_SHARED((N, H), jnp.float32),
        pltpu.VMEM((3, K), jnp.int32),
        pltpu.VMEM((3, K), jnp.int32),
        pltpu.VMEM((K, H), jnp.float32),
        pltpu.VMEM((K, H), jnp.float32),
        pltpu.VMEM((K, 16), jnp.float32),
        pltpu.VMEM((K, 16), jnp.float32),
        pltpu.SemaphoreType.DMA,
        pltpu.SemaphoreType.DMA,
        pltpu.SemaphoreType.DMA,
        pltpu.SemaphoreType.DMA,
    ],
)
def _agg_sc(q_hbm, sd5_hbm, ew16_hbm, zeros_hbm, out_hbm,
            acc_sh, idx_a, idx_b, rows_a, rows_b, ewb_a, ewb_b,
            isem_a, isem_b, gsem_a, gsem_b):
    cid = lax.axis_index("c")
    sid = lax.axis_index("s")
    wid = cid * NS + sid
    _copy_rows(sid, lambda o, n: zeros_hbm.at[pl.ds(o, n)],
               lambda o, n: acc_sh.at[pl.ds(o, n)])
    plsc.subcore_barrier()

    pltpu.async_copy(sd5_hbm.at[wid, 0], idx_a, isem_a)
    pltpu.async_copy(ew16_hbm.at[wid, 0], ewb_a, isem_a)
    pltpu.make_async_copy(sd5_hbm.at[wid, 0], idx_a, isem_a).wait()
    pltpu.make_async_copy(ew16_hbm.at[wid, 0], ewb_a, isem_a).wait()
    pltpu.async_copy(sd5_hbm.at[wid, 1], idx_b, isem_b)
    pltpu.async_copy(ew16_hbm.at[wid, 1], ewb_b, isem_b)
    pltpu.async_copy(q_hbm.at[idx_a.at[0]], rows_a, gsem_a)

    def scale_rows(ewb_cur, rows_cur):
        def row_body(r4, carry):
            for u in range(4):
                r = r4 * 4 + u
                w = ewb_cur[r]
                for g in range(NG):
                    v = rows_cur[r, pl.ds(g * 16, 16)]
                    rows_cur[r, pl.ds(g * 16, 16)] = v * w
            return carry

        lax.fori_loop(0, K // 4, row_body, 0)

    def step(ch, idx_cur, ewb_cur, isem_cur, gsem_cur, rows_cur,
             idx_nxt, ewb_nxt, isem_nxt, gsem_nxt, rows_nxt):
        pltpu.make_async_copy(q_hbm.at[idx_cur.at[0]], rows_cur,
                              gsem_cur).wait()

        @pl.when(ch + 1 < NCH)
        def _():
            pltpu.make_async_copy(sd5_hbm.at[wid, ch + 1], idx_nxt,
                                  isem_nxt).wait()
            pltpu.make_async_copy(ew16_hbm.at[wid, ch + 1], ewb_nxt,
                                  isem_nxt).wait()
            pltpu.async_copy(q_hbm.at[idx_nxt.at[0]], rows_nxt, gsem_nxt)

        scale_rows(ewb_cur, rows_cur)
        pltpu.sync_copy(rows_cur, acc_sh.at[idx_cur.at[1]], add=True)

        @pl.when(ch + 2 < NCH)
        def _():
            pltpu.async_copy(sd5_hbm.at[wid, ch + 2], idx_cur, isem_cur)
            pltpu.async_copy(ew16_hbm.at[wid, ch + 2], ewb_cur, isem_cur)

    def body(ch, carry):
        @pl.when(lax.rem(ch, 2) == 0)
        def _():
            step(ch, idx_a, ewb_a, isem_a, gsem_a, rows_a,
                 idx_b, ewb_b, isem_b, gsem_b, rows_b)

        @pl.when(lax.rem(ch, 2) == 1)
        def _():
            step(ch, idx_b, ewb_b, isem_b, gsem_b, rows_b,
                 idx_a, ewb_a, isem_a, gsem_a, rows_a)

        return carry

    lax.fori_loop(0, NCH, body, 0)
    plsc.subcore_barrier()
    _copy_rows(sid, lambda o, n: acc_sh.at[pl.ds(o, n)],
               lambda o, n: out_hbm.at[cid, pl.ds(o, n)])


@functools.partial(
    pl.kernel,
    out_type=jax.ShapeDtypeStruct((NW, NCH, K, 16), jnp.float32),
    mesh=_mesh,
    scratch_types=[
        pltpu.VMEM((3, K), jnp.int32),
        pltpu.VMEM((K, H), jnp.float32),
        pltpu.VMEM((K, H), jnp.float32),
        pltpu.VMEM((K, 16), jnp.float32),
        pltpu.SemaphoreType.DMA,
    ],
)
def _pew_sc(degsum_hbm, sd5_hbm, out_hbm, idx_v, rs_v, rd_v, p_v, sem):
    cid = lax.axis_index("c")
    sid = lax.axis_index("s")
    wid = cid * NS + sid
    one = jnp.full((16,), 1.0, jnp.float32)

    def chunk(ch, carry):
        pltpu.sync_copy(sd5_hbm.at[wid, ch], idx_v)
        pltpu.async_copy(degsum_hbm.at[idx_v.at[0]], rs_v, sem).wait()
        pltpu.async_copy(degsum_hbm.at[idx_v.at[1]], rd_v, sem).wait()

        def row_body(r, c):
            ds_ = jnp.maximum(rs_v[r, pl.ds(0, 16)], one)
            dd_ = jnp.maximum(rd_v[r, pl.ds(0, 16)], one)
            p_v[r] = ds_ * dd_
            return c

        lax.fori_loop(0, K, row_body, 0)
        pltpu.sync_copy(p_v, out_hbm.at[wid, ch])
        return carry

    lax.fori_loop(0, NCH, chunk, 0)


BN = 1000
GRID = N // BN


def _degsum_body(deg_ref, out_ref):
    out_ref[...] = deg_ref[0] + deg_ref[1]


_degsum_tc = pl.pallas_call(
    _degsum_body,
    grid=(GRID,),
    in_specs=[pl.BlockSpec((NC, BN, H), lambda i: (0, i, 0))],
    out_specs=pl.BlockSpec((BN, H), lambda i: (i, 0)),
    out_shape=jax.ShapeDtypeStruct((N, H), jnp.float32),
)


def _layer_body(h_ref, s_ref, ws_ref, wn_ref, b_ref, h_out_ref, *, act):
    sb = s_ref[0] + s_ref[1]
    z = (jnp.dot(h_ref[...], ws_ref[...], preferred_element_type=jnp.float32)
         + jnp.dot(sb, wn_ref[...], preferred_element_type=jnp.float32)
         + b_ref[...])
    if act:
        z = jnp.maximum(z, 0.0)
    h_out_ref[...] = z


def _make_layer_tc(act):
    return pl.pallas_call(
        functools.partial(_layer_body, act=act),
        grid=(GRID,),
        in_specs=[
            pl.BlockSpec((BN, H), lambda i: (i, 0)),
            pl.BlockSpec((NC, BN, H), lambda i: (0, i, 0)),
            pl.BlockSpec((H, H), lambda i: (0, 0)),
            pl.BlockSpec((H, H), lambda i: (0, 0)),
            pl.BlockSpec((1, H), lambda i: (0, 0)),
        ],
        out_specs=pl.BlockSpec((BN, H), lambda i: (i, 0)),
        out_shape=jax.ShapeDtypeStruct((N, H), jnp.float32),
    )


_layer_tc = _make_layer_tc(act=True)
_final_tc = _make_layer_tc(act=False)


def kernel(x, edge_index, W_in_self, W_in_neigh, b_in,
           W_mid_self, W_mid_neigh, b_mid,
           W_out_self, W_out_neigh, b_out):
    sd3 = (edge_index.astype(jnp.int32)
           .reshape(2, NW, NCH, K).transpose(1, 2, 0, 3))
    one_bits = jnp.full((NW, NCH, 1, K),
                        jnp.float32(1.0).view(jnp.int32), jnp.int32)
    sd5_ones = jnp.concatenate([sd3, one_bits], axis=2)
    onesNH = jnp.ones((N, H), jnp.float32)
    zerosH = jnp.zeros((N, H), jnp.float32)

    ones16e = jnp.ones((NW, NCH, K, 16), jnp.float32)
    deg2 = _agg_sc(onesNH, sd5_ones, ones16e, zerosH)
    degsum = _degsum_tc(deg2)
    p16 = _pew_sc(degsum, sd5_ones)
    ew16 = 1.0 / jnp.sqrt(p16)

    Wso = jnp.zeros((H, H), jnp.float32).at[:, :C].set(W_out_self)
    Wno = jnp.zeros((H, H), jnp.float32).at[:, :C].set(W_out_neigh)
    bo = jnp.zeros((1, H), jnp.float32).at[0, :C].set(b_out)

    h = x
    for li in range(L_MID + 2):
        s = _agg_sc(h, sd5_ones, ew16, zerosH)
        if li == 0:
            h = _layer_tc(h, s, W_in_self, W_in_neigh, b_in.reshape(1, H))
        elif li <= L_MID:
            h = _layer_tc(h, s, W_mid_self[li - 1], W_mid_neigh[li - 1],
                          b_mid[li - 1].reshape(1, H))
        else:
            h = _final_tc(h, s, Wso, Wno, bo)
    return h[:, :C]


# speedup vs baseline: 1.1645x; 1.1632x over previous
import functools

import jax
import jax.numpy as jnp
from jax import lax
from jax.experimental import pallas as pl
from jax.experimental.pallas import tpu as pltpu
from jax.experimental.pallas import tpu_sc as plsc

N = 10000
E = 320000
H = 128
C = 3
L_MID = 12

NC = 2
NS = 16
NW = NC * NS
EPW = E // NW
K = 80
NCH = EPW // K
RPW = 624
TAIL = N - NS * RPW
NG = H // 16

_mesh = plsc.VectorSubcoreMesh(core_axis_name="c", subcore_axis_name="s")


def _copy_rows(sid, get_src, get_dst):
    pltpu.sync_copy(get_src(sid * RPW, RPW), get_dst(sid * RPW, RPW))

    @pl.when(sid == NS - 1)
    def _():
        pltpu.sync_copy(get_src(NS * RPW, TAIL), get_dst(NS * RPW, TAIL))


@functools.partial(
    pl.kernel,
    out_type=jax.ShapeDtypeStruct((NC, N, H), jnp.float32),
    mesh=_mesh,
    scratch_types=[
        pltpu.VMEM_SHARED((N, H), jnp.float32),
        pltpu.VMEM((3, K), jnp.int32),
        pltpu.VMEM((3, K), jnp.int32),
        pltpu.VMEM((K, H), jnp.float32),
        pltpu.VMEM((K, H), jnp.float32),
        pltpu.VMEM((K, 16), jnp.float32),
        pltpu.VMEM((K, 16), jnp.float32),
        pltpu.VMEM((K,), jnp.int32),
        pltpu.VMEM((K,), jnp.int32),
        pltpu.SemaphoreType.DMA,
        pltpu.SemaphoreType.DMA,
        pltpu.SemaphoreType.DMA,
        pltpu.SemaphoreType.DMA,
        pltpu.SemaphoreType.DMA,
        pltpu.SemaphoreType.DMA,
    ],
)
def _agg_sc(q_hbm, sd5_hbm, ew16_hbm, zeros_hbm, out_hbm,
            acc_sh, idx_a, idx_b, rows_a, rows_b, ewb_a, ewb_b,
            sidx_a, sidx_b, isem_a, isem_b, gsem_a, gsem_b, ssem_a, ssem_b):
    cid = lax.axis_index("c")
    sid = lax.axis_index("s")
    wid = cid * NS + sid
    _copy_rows(sid, lambda o, n: zeros_hbm.at[pl.ds(o, n)],
               lambda o, n: acc_sh.at[pl.ds(o, n)])
    plsc.subcore_barrier()

    pltpu.async_copy(sd5_hbm.at[wid, 0], idx_a, isem_a)
    pltpu.async_copy(ew16_hbm.at[wid, 0], ewb_a, isem_a)
    pltpu.make_async_copy(sd5_hbm.at[wid, 0], idx_a, isem_a).wait()
    pltpu.make_async_copy(ew16_hbm.at[wid, 0], ewb_a, isem_a).wait()
    pltpu.async_copy(sd5_hbm.at[wid, 1], idx_b, isem_b)
    pltpu.async_copy(ew16_hbm.at[wid, 1], ewb_b, isem_b)
    pltpu.async_copy(q_hbm.at[idx_a.at[0]], rows_a, gsem_a)

    def scale_rows(ewb_cur, rows_cur):
        def row_body(r4, carry):
            for u in range(4):
                r = r4 * 4 + u
                w = ewb_cur[r]
                for g in range(NG):
                    v = rows_cur[r, pl.ds(g * 16, 16)]
                    rows_cur[r, pl.ds(g * 16, 16)] = v * w
            return carry

        lax.fori_loop(0, K // 4, row_body, 0)

    def step(ch, idx_cur, ewb_cur, sidx_cur, isem_cur, gsem_cur, ssem_cur,
             rows_cur, idx_nxt, ewb_nxt, sidx_nxt, isem_nxt, gsem_nxt,
             ssem_nxt, rows_nxt):
        pltpu.make_async_copy(q_hbm.at[idx_cur.at[0]], rows_cur,
                              gsem_cur).wait()

        @pl.when(ch + 1 < NCH)
        def _():
            @pl.when(ch >= 1)
            def _():
                pltpu.make_async_copy(rows_nxt, acc_sh.at[sidx_nxt],
                                      ssem_nxt).wait()

            pltpu.make_async_copy(sd5_hbm.at[wid, ch + 1], idx_nxt,
                                  isem_nxt).wait()
            pltpu.make_async_copy(ew16_hbm.at[wid, ch + 1], ewb_nxt,
                                  isem_nxt).wait()
            pltpu.async_copy(q_hbm.at[idx_nxt.at[0]], rows_nxt, gsem_nxt)

        scale_rows(ewb_cur, rows_cur)
        for j in range(K // 16):
            sidx_cur[pl.ds(j * 16, 16)] = idx_cur[1, pl.ds(j * 16, 16)]
        pltpu.async_copy(rows_cur, acc_sh.at[sidx_cur], ssem_cur, add=True)

        @pl.when(ch + 2 < NCH)
        def _():
            pltpu.async_copy(sd5_hbm.at[wid, ch + 2], idx_cur, isem_cur)
            pltpu.async_copy(ew16_hbm.at[wid, ch + 2], ewb_cur, isem_cur)

    def body(ch, carry):
        @pl.when(lax.rem(ch, 2) == 0)
        def _():
            step(ch, idx_a, ewb_a, sidx_a, isem_a, gsem_a, ssem_a, rows_a,
                 idx_b, ewb_b, sidx_b, isem_b, gsem_b, ssem_b, rows_b)

        @pl.when(lax.rem(ch, 2) == 1)
        def _():
            step(ch, idx_b, ewb_b, sidx_b, isem_b, gsem_b, ssem_b, rows_b,
                 idx_a, ewb_a, sidx_a, isem_a, gsem_a, ssem_a, rows_a)

        return carry

    lax.fori_loop(0, NCH, body, 0)
    pltpu.make_async_copy(rows_b, acc_sh.at[sidx_b], ssem_b).wait()
    pltpu.make_async_copy(rows_a, acc_sh.at[sidx_a], ssem_a).wait()
    plsc.subcore_barrier()
    _copy_rows(sid, lambda o, n: acc_sh.at[pl.ds(o, n)],
               lambda o, n: out_hbm.at[cid, pl.ds(o, n)])


@functools.partial(
    pl.kernel,
    out_type=jax.ShapeDtypeStruct((NW, NCH, K, 16), jnp.float32),
    mesh=_mesh,
    scratch_types=[
        pltpu.VMEM((3, K), jnp.int32),
        pltpu.VMEM((K, H), jnp.float32),
        pltpu.VMEM((K, H), jnp.float32),
        pltpu.VMEM((K, 16), jnp.float32),
        pltpu.SemaphoreType.DMA,
    ],
)
def _pew_sc(degsum_hbm, sd5_hbm, out_hbm, idx_v, rs_v, rd_v, p_v, sem):
    cid = lax.axis_index("c")
    sid = lax.axis_index("s")
    wid = cid * NS + sid
    one = jnp.full((16,), 1.0, jnp.float32)

    def chunk(ch, carry):
        pltpu.sync_copy(sd5_hbm.at[wid, ch], idx_v)
        pltpu.async_copy(degsum_hbm.at[idx_v.at[0]], rs_v, sem).wait()
        pltpu.async_copy(degsum_hbm.at[idx_v.at[1]], rd_v, sem).wait()

        def row_body(r, c):
            ds_ = jnp.maximum(rs_v[r, pl.ds(0, 16)], one)
            dd_ = jnp.maximum(rd_v[r, pl.ds(0, 16)], one)
            p_v[r] = ds_ * dd_
            return c

        lax.fori_loop(0, K, row_body, 0)
        pltpu.sync_copy(p_v, out_hbm.at[wid, ch])
        return carry

    lax.fori_loop(0, NCH, chunk, 0)


BN = 1000
GRID = N // BN


def _degsum_body(deg_ref, out_ref):
    out_ref[...] = deg_ref[0] + deg_ref[1]


_degsum_tc = pl.pallas_call(
    _degsum_body,
    grid=(GRID,),
    in_specs=[pl.BlockSpec((NC, BN, H), lambda i: (0, i, 0))],
    out_specs=pl.BlockSpec((BN, H), lambda i: (i, 0)),
    out_shape=jax.ShapeDtypeStruct((N, H), jnp.float32),
)


def _layer_body(h_ref, s_ref, ws_ref, wn_ref, b_ref, h_out_ref, *, act):
    sb = s_ref[0] + s_ref[1]
    z = (jnp.dot(h_ref[...], ws_ref[...], preferred_element_type=jnp.float32)
         + jnp.dot(sb, wn_ref[...], preferred_element_type=jnp.float32)
         + b_ref[...])
    if act:
        z = jnp.maximum(z, 0.0)
    h_out_ref[...] = z


def _make_layer_tc(act):
    return pl.pallas_call(
        functools.partial(_layer_body, act=act),
        grid=(GRID,),
        in_specs=[
            pl.BlockSpec((BN, H), lambda i: (i, 0)),
            pl.BlockSpec((NC, BN, H), lambda i: (0, i, 0)),
            pl.BlockSpec((H, H), lambda i: (0, 0)),
            pl.BlockSpec((H, H), lambda i: (0, 0)),
            pl.BlockSpec((1, H), lambda i: (0, 0)),
        ],
        out_specs=pl.BlockSpec((BN, H), lambda i: (i, 0)),
        out_shape=jax.ShapeDtypeStruct((N, H), jnp.float32),
    )


_layer_tc = _make_layer_tc(act=True)
_final_tc = _make_layer_tc(act=False)


def kernel(x, edge_index, W_in_self, W_in_neigh, b_in,
           W_mid_self, W_mid_neigh, b_mid,
           W_out_self, W_out_neigh, b_out):
    sd3 = (edge_index.astype(jnp.int32)
           .reshape(2, NW, NCH, K).transpose(1, 2, 0, 3))
    one_bits = jnp.full((NW, NCH, 1, K),
                        jnp.float32(1.0).view(jnp.int32), jnp.int32)
    sd5_ones = jnp.concatenate([sd3, one_bits], axis=2)
    onesNH = jnp.ones((N, H), jnp.float32)
    zerosH = jnp.zeros((N, H), jnp.float32)

    ones16e = jnp.ones((NW, NCH, K, 16), jnp.float32)
    deg2 = _agg_sc(onesNH, sd5_ones, ones16e, zerosH)
    degsum = _degsum_tc(deg2)
    p16 = _pew_sc(degsum, sd5_ones)
    ew16 = 1.0 / jnp.sqrt(p16)

    Wso = jnp.zeros((H, H), jnp.float32).at[:, :C].set(W_out_self)
    Wno = jnp.zeros((H, H), jnp.float32).at[:, :C].set(W_out_neigh)
    bo = jnp.zeros((1, H), jnp.float32).at[0, :C].set(b_out)

    h = x
    for li in range(L_MID + 2):
        s = _agg_sc(h, sd5_ones, ew16, zerosH)
        if li == 0:
            h = _layer_tc(h, s, W_in_self, W_in_neigh, b_in.reshape(1, H))
        elif li <= L_MID:
            h = _layer_tc(h, s, W_mid_self[li - 1], W_mid_neigh[li - 1],
                          b_mid[li - 1].reshape(1, H))
        else:
            h = _final_tc(h, s, Wso, Wno, bo)
    return h[:, :C]
